# Initial kernel scaffold; baseline (speedup 1.0000x reference)
import jax
import jax.numpy as jnp
from jax.experimental import pallas as pl


def kernel(des, tweet, num_prop, cat_prop, edge_index, edge_type, W_in, b_in, Wr, Wroot, b_rgcn, Wg, W1, b1, W2, b2):
    raise NotImplementedError("write your pallas kernel here")


# speedup vs baseline: 1.2578x; 1.2578x over previous
import functools

import jax
import jax.numpy as jnp
from jax.experimental import pallas as pl
from jax.experimental.pallas import tpu as pltpu

D = 256
NEXP = 8
OUT = 2
NREL = 2
BN = 512
ECHUNK = 4000


def _dot(a, b):
    return jnp.dot(a, b, preferred_element_type=jnp.float32)


def _leaky_relu(x):
    return jnp.where(x >= 0, x, 0.01 * x)


def _proj_body(x_ref, w_ref, b_ref, o_ref):
    o_ref[...] = _dot(x_ref[...], w_ref[...]) + b_ref[...]


def _proj(x0, W_in, b_in):
    npad = x0.shape[0]
    return pl.pallas_call(
        _proj_body,
        grid=(npad // BN,),
        in_specs=[
            pl.BlockSpec((BN, D), lambda i: (i, 0)),
            pl.BlockSpec((D, D), lambda i: (0, 0)),
            pl.BlockSpec((1, D), lambda i: (0, 0)),
        ],
        out_specs=pl.BlockSpec((BN, D), lambda i: (i, 0)),
        out_shape=jax.ShapeDtypeStruct((npad, D), jnp.float32),
    )(x0, W_in, b_in.reshape(1, D))


def _agg_body(src_ref, key_ref, x_ref, agg_ref, cnt_ref):
    @pl.when(pl.program_id(0) == 0)
    def _init():
        agg_ref[...] = jnp.zeros_like(agg_ref)
        cnt_ref[...] = jnp.zeros_like(cnt_ref)

    def step(e, carry):
        s = src_ref[0, 0, e]
        k = key_ref[0, 0, e]
        agg_ref[pl.ds(k, 1), :] += x_ref[pl.ds(s, 1), :]
        cnt_ref[pl.ds(k, 1), :] += 1.0
        return carry

    jax.lax.fori_loop(0, ECHUNK, step, 0)


def _aggregate(x, src_c, key_c, nrows):
    nchunks = src_c.shape[0]
    agg, cnt = pl.pallas_call(
        _agg_body,
        grid=(nchunks,),
        in_specs=[
            pl.BlockSpec((1, 1, ECHUNK), lambda i: (i, 0, 0),
                         memory_space=pltpu.SMEM),
            pl.BlockSpec((1, 1, ECHUNK), lambda i: (i, 0, 0),
                         memory_space=pltpu.SMEM),
            pl.BlockSpec((x.shape[0], D), lambda i: (0, 0)),
        ],
        out_specs=[
            pl.BlockSpec((nrows, D), lambda i: (0, 0)),
            pl.BlockSpec((nrows, 1), lambda i: (0, 0)),
        ],
        out_shape=[
            jax.ShapeDtypeStruct((nrows, D), jnp.float32),
            jax.ShapeDtypeStruct((nrows, 1), jnp.float32),
        ],
    )(src_c, key_c, x)
    return agg, cnt


def _combine_body(x_ref, a0_ref, a1_ref, c0_ref, c1_ref,
                  wroot_ref, w0_ref, w1_ref, b_ref, o_ref):
    acc = _dot(x_ref[...], wroot_ref[...]) + b_ref[...]
    m0 = a0_ref[...] / jnp.maximum(c0_ref[...], 1.0)
    m1 = a1_ref[...] / jnp.maximum(c1_ref[...], 1.0)
    acc += _dot(m0, w0_ref[...])
    acc += _dot(m1, w1_ref[...])
    o_ref[...] = acc


def _combine(x, agg0, agg1, cnt0, cnt1, Wroot, Wr, b_rgcn):
    npad = x.shape[0]
    return pl.pallas_call(
        _combine_body,
        grid=(npad // BN,),
        in_specs=[
            pl.BlockSpec((BN, D), lambda i: (i, 0)),
            pl.BlockSpec((BN, D), lambda i: (i, 0)),
            pl.BlockSpec((BN, D), lambda i: (i, 0)),
            pl.BlockSpec((BN, 1), lambda i: (i, 0)),
            pl.BlockSpec((BN, 1), lambda i: (i, 0)),
            pl.BlockSpec((D, D), lambda i: (0, 0)),
            pl.BlockSpec((D, D), lambda i: (0, 0)),
            pl.BlockSpec((D, D), lambda i: (0, 0)),
            pl.BlockSpec((1, D), lambda i: (0, 0)),
        ],
        out_specs=pl.BlockSpec((BN, D), lambda i: (i, 0)),
        out_shape=jax.ShapeDtypeStruct((npad, D), jnp.float32),
    )(x, agg0, agg1, cnt0, cnt1, Wroot, Wr[0], Wr[1],
      b_rgcn.reshape(1, D))


def _moe_body(x_ref, wg_ref, w1_ref, b1_ref, w2_ref, b2_ref, o_ref):
    x = x_ref[...]
    logits = _dot(x, wg_ref[...])
    ids = jax.lax.broadcasted_iota(jnp.int32, logits.shape, 1)
    m1 = jnp.max(logits, axis=1, keepdims=True)
    i1 = jnp.argmax(logits, axis=1)[:, None]
    masked = jnp.where(ids == i1, -jnp.inf, logits)
    m2 = jnp.max(masked, axis=1, keepdims=True)
    i2 = jnp.argmax(masked, axis=1)[:, None]
    t = jnp.exp(m2 - m1)
    den = 1.0 + t
    g1 = 1.0 / den
    g2 = t / den
    acc = jnp.zeros((x.shape[0], OUT), jnp.float32)
    for e in range(NEXP):
        h = _leaky_relu(
            _dot(x, w1_ref[e])
            + b1_ref[...][e][None, :])
        y = _dot(h, w2_ref[e]) \
            + b2_ref[...][e][None, :]
        w = jnp.where(i1 == e, g1, 0.0) + jnp.where(i2 == e, g2, 0.0)
        acc += w * y
    o_ref[...] = acc


def _moe(x, Wg, W1, b1, W2, b2):
    npad = x.shape[0]
    return pl.pallas_call(
        _moe_body,
        grid=(npad // BN,),
        in_specs=[
            pl.BlockSpec((BN, D), lambda i: (i, 0)),
            pl.BlockSpec((D, NEXP), lambda i: (0, 0)),
            pl.BlockSpec((NEXP, D, D), lambda i: (0, 0, 0)),
            pl.BlockSpec((NEXP, D), lambda i: (0, 0)),
            pl.BlockSpec((NEXP, D, OUT), lambda i: (0, 0, 0)),
            pl.BlockSpec((NEXP, OUT), lambda i: (0, 0)),
        ],
        out_specs=pl.BlockSpec((BN, OUT), lambda i: (i, 0)),
        out_shape=jax.ShapeDtypeStruct((npad, OUT), jnp.float32),
    )(x, Wg, W1, b1, W2, b2)


def kernel(des, tweet, num_prop, cat_prop, edge_index, edge_type,
           W_in, b_in, Wr, Wroot, b_rgcn, Wg, W1, b1, W2, b2):
    n = des.shape[0]
    e = edge_index.shape[1]
    npad = ((n + BN - 1) // BN) * BN
    nrows = NREL * npad + 8
    epad = ((e + ECHUNK - 1) // ECHUNK) * ECHUNK
    nchunks = epad // ECHUNK

    x0 = jnp.concatenate([des, tweet, num_prop, cat_prop], axis=1)
    x0 = jnp.pad(x0, ((0, npad - n), (0, 0)))

    src = edge_index[0]
    dst = edge_index[1]
    key = edge_type * npad + dst
    src = jnp.pad(src, (0, epad - e))
    key = jnp.pad(key, (0, epad - e), constant_values=NREL * npad)
    src_c = src.reshape(nchunks, 1, ECHUNK)
    key_c = key.reshape(nchunks, 1, ECHUNK)

    x1 = jax.nn.selu(_proj(x0, W_in, b_in))

    agg, cnt = _aggregate(x1, src_c, key_c, nrows)
    x2 = _combine(x1, agg[:npad], agg[npad:2 * npad],
                  cnt[:npad], cnt[npad:2 * npad], Wroot, Wr, b_rgcn)

    agg2, cnt2 = _aggregate(x2, src_c, key_c, nrows)
    x3 = _combine(x2, agg2[:npad], agg2[npad:2 * npad],
                  cnt2[:npad], cnt2[npad:2 * npad], Wroot, Wr, b_rgcn)

    out = _moe(x3, Wg, W1, b1, W2, b2)
    return out[:n]


# speedup vs baseline: 1.2603x; 1.0020x over previous
import functools

import jax
import jax.numpy as jnp
from jax.experimental import pallas as pl
from jax.experimental.pallas import tpu as pltpu

D = 256
NEXP = 8
OUT = 2
NREL = 2
BN = 512
ECHUNK = 4000


def _dot(a, b):
    return jnp.dot(a, b, preferred_element_type=jnp.float32)


def _leaky_relu(x):
    return jnp.where(x >= 0, x, 0.01 * x)


def _proj_body(x_ref, w_ref, b_ref, o_ref):
    o_ref[...] = _dot(x_ref[...], w_ref[...]) + b_ref[...]


def _proj(x0, W_in, b_in):
    npad = x0.shape[0]
    return pl.pallas_call(
        _proj_body,
        grid=(npad // BN,),
        in_specs=[
            pl.BlockSpec((BN, D), lambda i: (i, 0)),
            pl.BlockSpec((D, D), lambda i: (0, 0)),
            pl.BlockSpec((1, D), lambda i: (0, 0)),
        ],
        out_specs=pl.BlockSpec((BN, D), lambda i: (i, 0)),
        out_shape=jax.ShapeDtypeStruct((npad, D), jnp.float32),
    )(x0, W_in, b_in.reshape(1, D))


def _agg_body(src_ref, key_ref, x_ref, agg_ref, cnt_ref):
    @pl.when(pl.program_id(0) == 0)
    def _init():
        agg_ref[...] = jnp.zeros_like(agg_ref)
        cnt_ref[...] = jnp.zeros_like(cnt_ref)

    def step(e, carry):
        s = src_ref[0, 0, e]
        k = key_ref[0, 0, e]
        agg_ref[pl.ds(k, 1), :] += x_ref[pl.ds(s, 1), :]
        cnt_ref[pl.ds(k, 1), :] += 1.0
        return carry

    jax.lax.fori_loop(0, ECHUNK, step, 0)


def _aggregate(x, src_c, key_c, nrows):
    nchunks = src_c.shape[0]
    agg, cnt = pl.pallas_call(
        _agg_body,
        grid=(nchunks,),
        in_specs=[
            pl.BlockSpec((1, 1, ECHUNK), lambda i: (i, 0, 0),
                         memory_space=pltpu.SMEM),
            pl.BlockSpec((1, 1, ECHUNK), lambda i: (i, 0, 0),
                         memory_space=pltpu.SMEM),
            pl.BlockSpec((x.shape[0], D), lambda i: (0, 0)),
        ],
        out_specs=[
            pl.BlockSpec((nrows, D), lambda i: (0, 0)),
            pl.BlockSpec((nrows, 1), lambda i: (0, 0)),
        ],
        out_shape=[
            jax.ShapeDtypeStruct((nrows, D), jnp.float32),
            jax.ShapeDtypeStruct((nrows, 1), jnp.float32),
        ],
    )(src_c, key_c, x)
    return agg, cnt


def _combine_body(x_ref, a0_ref, a1_ref, c0_ref, c1_ref,
                  wroot_ref, w0_ref, w1_ref, b_ref, o_ref):
    acc = _dot(x_ref[...], wroot_ref[...]) + b_ref[...]
    m0 = a0_ref[...] / jnp.maximum(c0_ref[...], 1.0)
    m1 = a1_ref[...] / jnp.maximum(c1_ref[...], 1.0)
    acc += _dot(m0, w0_ref[...])
    acc += _dot(m1, w1_ref[...])
    o_ref[...] = acc


def _combine(x, agg0, agg1, cnt0, cnt1, Wroot, Wr, b_rgcn):
    npad = x.shape[0]
    return pl.pallas_call(
        _combine_body,
        grid=(npad // BN,),
        in_specs=[
            pl.BlockSpec((BN, D), lambda i: (i, 0)),
            pl.BlockSpec((BN, D), lambda i: (i, 0)),
            pl.BlockSpec((BN, D), lambda i: (i, 0)),
            pl.BlockSpec((BN, 1), lambda i: (i, 0)),
            pl.BlockSpec((BN, 1), lambda i: (i, 0)),
            pl.BlockSpec((D, D), lambda i: (0, 0)),
            pl.BlockSpec((D, D), lambda i: (0, 0)),
            pl.BlockSpec((D, D), lambda i: (0, 0)),
            pl.BlockSpec((1, D), lambda i: (0, 0)),
        ],
        out_specs=pl.BlockSpec((BN, D), lambda i: (i, 0)),
        out_shape=jax.ShapeDtypeStruct((npad, D), jnp.float32),
    )(x, agg0, agg1, cnt0, cnt1, Wroot, Wr[0], Wr[1],
      b_rgcn.reshape(1, D))


def _moe_body(x_ref, wg_ref, w1_ref, b1_ref, w2_ref, b2_ref, o_ref):
    x = x_ref[...]
    logits = _dot(x, wg_ref[...])
    ids = jax.lax.broadcasted_iota(jnp.int32, logits.shape, 1)
    m1 = jnp.max(logits, axis=1, keepdims=True)
    i1 = jnp.argmax(logits, axis=1)[:, None]
    masked = jnp.where(ids == i1, -jnp.inf, logits)
    m2 = jnp.max(masked, axis=1, keepdims=True)
    i2 = jnp.argmax(masked, axis=1)[:, None]
    t = jnp.exp(m2 - m1)
    den = 1.0 + t
    g1 = 1.0 / den
    g2 = t / den
    acc = jnp.zeros((x.shape[0], OUT), jnp.float32)
    for e in range(NEXP):
        h = _leaky_relu(
            _dot(x, w1_ref[e])
            + b1_ref[...][e][None, :])
        y = _dot(h, w2_ref[e]) \
            + b2_ref[...][e][None, :]
        w = jnp.where(i1 == e, g1, 0.0) + jnp.where(i2 == e, g2, 0.0)
        acc += w * y
    o_ref[...] = acc


def _moe(x, Wg, W1, b1, W2, b2):
    npad = x.shape[0]
    return pl.pallas_call(
        _moe_body,
        grid=(npad // BN,),
        in_specs=[
            pl.BlockSpec((BN, D), lambda i: (i, 0)),
            pl.BlockSpec((D, NEXP), lambda i: (0, 0)),
            pl.BlockSpec((NEXP, D, D), lambda i: (0, 0, 0)),
            pl.BlockSpec((NEXP, D), lambda i: (0, 0)),
            pl.BlockSpec((NEXP, D, OUT), lambda i: (0, 0, 0)),
            pl.BlockSpec((NEXP, OUT), lambda i: (0, 0)),
        ],
        out_specs=pl.BlockSpec((BN, OUT), lambda i: (i, 0)),
        out_shape=jax.ShapeDtypeStruct((npad, OUT), jnp.float32),
    )(x, Wg, W1, b1, W2, b2)


def kernel(des, tweet, num_prop, cat_prop, edge_index, edge_type,
           W_in, b_in, Wr, Wroot, b_rgcn, Wg, W1, b1, W2, b2):
    n = des.shape[0]
    e = edge_index.shape[1]
    npad = ((n + BN - 1) // BN) * BN
    nh = npad // 2
    nha = nh + 128
    epad = ((e + 63) // 64) * 64

    x0 = jnp.concatenate([des, tweet, num_prop, cat_prop], axis=1)
    x0 = jnp.pad(x0, ((0, npad - n), (0, 0)))

    epad = ((e + ECHUNK - 1) // ECHUNK) * ECHUNK
    nchunks = epad // ECHUNK
    nrows = NREL * npad + 8

    src = jnp.pad(edge_index[0], (0, epad - e))
    key = jnp.pad(edge_type * npad + edge_index[1], (0, epad - e),
                  constant_values=NREL * npad)
    src_c = src.reshape(nchunks, 1, ECHUNK)
    key_c = key.reshape(nchunks, 1, ECHUNK)

    x1 = jax.nn.selu(_proj(x0, W_in, b_in))

    agg, cnt = _aggregate(x1, src_c, key_c, nrows)
    x2 = _combine(x1, agg[:npad], agg[npad:2 * npad],
                  cnt[:npad], cnt[npad:2 * npad], Wroot, Wr, b_rgcn)

    agg2, _ = _aggregate(x2, src_c, key_c, nrows)
    x3 = _combine(x2, agg2[:npad], agg2[npad:2 * npad],
                  cnt[:npad], cnt[npad:2 * npad], Wroot, Wr, b_rgcn)

    out = _moe(x3, Wg, W1, b1, W2, b2)
    return out[:n]
